# Initial kernel scaffold; baseline (speedup 1.0000x reference)
import jax
import jax.numpy as jnp
from jax.experimental import pallas as pl


def kernel(x1, edge_index1, x2, edge_index2, W_init, b_init, W1, b1, W2, b2, bn1_g, bn1_b, bn2_g, bn2_b, W_fc, b_fc, W_fc2, b_fc2):
    raise NotImplementedError("write your pallas kernel here")


# speedup vs baseline: 2.9317x; 2.9317x over previous
import functools

import jax
import jax.numpy as jnp
from jax import lax
from jax.experimental import pallas as pl
from jax.experimental.pallas import tpu as pltpu
from jax.experimental.pallas import tpu_sc as plsc

N = 10000
E = 320000
DIN = 128
H = 128

_NC = 2
_NS = 16
_C = 128
_K = 160
_EP = _NS * _K * _C
_NPAD = 10112
_RPT = _NPAD // _NS
_ORT = 624
_TAIL = N - _NS * _ORT

_sc_mesh = plsc.VectorSubcoreMesh(core_axis_name="c", subcore_axis_name="s")


def _sc_deg_body(dst_hbm, zacc_hbm, ones_hbm, deg_hbm,
                 dst_v, ones_v, acc_sh):
    c = lax.axis_index("c")
    s = lax.axis_index("s")
    tile = c * _NS + s
    pltpu.sync_copy(zacc_hbm, acc_sh.at[pl.ds(s * _RPT, _RPT)])
    pltpu.sync_copy(ones_hbm, ones_v)
    plsc.subcore_barrier()

    def step(j, carry):
        base = tile * _K * _C + j * _C
        pltpu.sync_copy(dst_hbm.at[pl.ds(base, _C)], dst_v)
        pltpu.sync_copy(ones_v, acc_sh.at[dst_v], add=True)
        return carry

    lax.fori_loop(0, _K, step, 0)
    plsc.subcore_barrier()
    base = c * N + s * _ORT
    pltpu.sync_copy(acc_sh.at[pl.ds(s * _ORT, _ORT)],
                    deg_hbm.at[pl.ds(base, _ORT)])

    @pl.when(s == _NS - 1)
    def _copy_tail():
        pltpu.sync_copy(acc_sh.at[pl.ds(_NS * _ORT, _TAIL)],
                        deg_hbm.at[pl.ds(c * N + _NS * _ORT, _TAIL)])


def _sc_agg_body(y_hbm, src_hbm, dst_hbm, zacc_hbm,
                 out_hbm,
                 src_v, dst_v, rows_v, acc_sh, sem):
    c = lax.axis_index("c")
    s = lax.axis_index("s")
    tile = c * _NS + s
    pltpu.sync_copy(zacc_hbm, acc_sh.at[pl.ds(s * _RPT, _RPT)])
    plsc.subcore_barrier()

    def step(j, carry):
        base = tile * _K * _C + j * _C
        pltpu.sync_copy(src_hbm.at[pl.ds(base, _C)], src_v)
        pltpu.sync_copy(dst_hbm.at[pl.ds(base, _C)], dst_v)
        pltpu.async_copy(y_hbm.at[src_v], rows_v, sem).wait()
        pltpu.sync_copy(rows_v, acc_sh.at[dst_v], add=True)
        return carry

    lax.fori_loop(0, _K, step, 0)
    plsc.subcore_barrier()
    base = c * N + s * _ORT
    pltpu.sync_copy(acc_sh.at[pl.ds(s * _ORT, _ORT)],
                    out_hbm.at[pl.ds(base, _ORT)])

    @pl.when(s == _NS - 1)
    def _copy_tail():
        pltpu.sync_copy(acc_sh.at[pl.ds(_NS * _ORT, _TAIL)],
                        out_hbm.at[pl.ds(c * N + _NS * _ORT, _TAIL)])


_sc_deg = pl.kernel(
    _sc_deg_body,
    out_type=jax.ShapeDtypeStruct((2 * N, H), jnp.float32),
    mesh=_sc_mesh,
    scratch_types=[
        pltpu.VMEM((_C,), jnp.int32),
        pltpu.VMEM((_C, H), jnp.float32),
        pltpu.VMEM_SHARED((_NPAD, H), jnp.float32),
    ],
)

_sc_agg = pl.kernel(
    _sc_agg_body,
    out_type=jax.ShapeDtypeStruct((2 * N, H), jnp.float32),
    mesh=_sc_mesh,
    scratch_types=[
        pltpu.VMEM((_C,), jnp.int32),
        pltpu.VMEM((_C,), jnp.int32),
        pltpu.VMEM((_C, H), jnp.float32),
        pltpu.VMEM_SHARED((_NPAD, H), jnp.float32),
        pltpu.SemaphoreType.DMA,
    ],
)


def _tc1_body(x_ref, wi_ref, bi_ref, w1_ref, y_ref):
    h0 = jnp.dot(x_ref[...], wi_ref[...],
                 preferred_element_type=jnp.float32) + bi_ref[...]
    y_ref[...] = jnp.dot(h0, w1_ref[...], preferred_element_type=jnp.float32)


def _tc1(x_all, w_init, b_init, w1):
    blk = 2000
    grid = (2 * N) // blk
    return pl.pallas_call(
        _tc1_body,
        grid=(grid,),
        in_specs=[
            pl.BlockSpec((blk, DIN), lambda g: (g, 0)),
            pl.BlockSpec((DIN, H), lambda g: (0, 0)),
            pl.BlockSpec((1, H), lambda g: (0, 0)),
            pl.BlockSpec((H, H), lambda g: (0, 0)),
        ],
        out_specs=pl.BlockSpec((blk, H), lambda g: (g, 0)),
        out_shape=jax.ShapeDtypeStruct((2 * N, H), jnp.float32),
    )(x_all, w_init, b_init, w1)


def _bn_relu(z, gamma, beta):
    m = jnp.mean(z, axis=0, keepdims=True)
    v = jnp.mean((z - m) ** 2, axis=0, keepdims=True)
    return jnp.maximum(gamma * (z - m) * lax.rsqrt(v + 1e-5) + beta, 0.0)


def _tc2_body(y_ref, s_ref, deg_ref, b_ref, g_ref, bb_ref, w2_ref, y2_ref):
    inv = 1.0 / jnp.maximum(deg_ref[0, :, 0:1], 1.0)
    z = y_ref[0] + s_ref[0] * inv + b_ref[...]
    h = _bn_relu(z, g_ref[...], bb_ref[...])
    y2_ref[0] = jnp.dot(h, w2_ref[...], preferred_element_type=jnp.float32)


def _tc2(y1, s1, deg, b1, g1, bb1, w2):
    return pl.pallas_call(
        _tc2_body,
        grid=(2,),
        in_specs=[
            pl.BlockSpec((1, N, H), lambda g: (g, 0, 0)),
            pl.BlockSpec((1, N, H), lambda g: (g, 0, 0)),
            pl.BlockSpec((1, N, H), lambda g: (g, 0, 0)),
            pl.BlockSpec((1, H), lambda g: (0, 0)),
            pl.BlockSpec((1, H), lambda g: (0, 0)),
            pl.BlockSpec((1, H), lambda g: (0, 0)),
            pl.BlockSpec((H, H), lambda g: (0, 0)),
        ],
        out_specs=pl.BlockSpec((1, N, H), lambda g: (g, 0, 0)),
        out_shape=jax.ShapeDtypeStruct((2, N, H), jnp.float32),
    )(y1, s1, deg, b1, g1, bb1, w2)


def _tc3_body(y_ref, s_ref, deg_ref, b_ref, g_ref, bb_ref,
              wfc_ref, bfc_ref, wfc2_ref, bfc2_ref, out_ref):
    embs = []
    for g in range(2):
        inv = 1.0 / jnp.maximum(deg_ref[g, :, 0:1], 1.0)
        z = y_ref[g] + s_ref[g] * inv + b_ref[...]
        h = _bn_relu(z, g_ref[...], bb_ref[...])
        embs.append(jnp.mean(h, axis=0, keepdims=True))
    cat = jnp.concatenate(embs, axis=1)
    t = jnp.maximum(jnp.dot(cat, wfc_ref[...],
                            preferred_element_type=jnp.float32) + bfc_ref[...],
                    0.0)
    out_ref[...] = jax.nn.sigmoid(
        jnp.dot(t, wfc2_ref[...], preferred_element_type=jnp.float32)
        + bfc2_ref[...])


def _tc3(y2, s2, deg, b2, g2, bb2, w_fc, b_fc, w_fc2, b_fc2):
    return pl.pallas_call(
        _tc3_body,
        out_shape=jax.ShapeDtypeStruct((1, 1), jnp.float32),
    )(y2, s2, deg, b2, g2, bb2, w_fc, b_fc, w_fc2, b_fc2)


def _prep_edges(edge_index, offset):
    src = edge_index[0]
    dst = edge_index[1]
    pad = _EP - E
    srcp = jnp.concatenate([src + offset, jnp.full((pad,), offset, jnp.int32)])
    dstp = jnp.concatenate([dst, jnp.full((pad,), N, jnp.int32)])
    return srcp, dstp


def kernel(x1, edge_index1, x2, edge_index2, W_init, b_init, W1, b1, W2, b2,
           bn1_g, bn1_b, bn2_g, bn2_b, W_fc, b_fc, W_fc2, b_fc2):
    x_all = jnp.concatenate([x1, x2], axis=0)
    src1, dst1 = _prep_edges(edge_index1, 0)
    src2, dst2 = _prep_edges(edge_index2, N)
    src_all = jnp.concatenate([src1, src2])
    dst_all = jnp.concatenate([dst1, dst2])

    zacc = jnp.zeros((_RPT, H), jnp.float32)
    ones = jnp.ones((_C, H), jnp.float32)

    b_init_r = b_init[None, :]
    b1_r = b1[None, :]
    b2_r = b2[None, :]
    bn1_g_r = bn1_g[None, :]
    bn1_b_r = bn1_b[None, :]
    bn2_g_r = bn2_g[None, :]
    bn2_b_r = bn2_b[None, :]
    b_fc_r = b_fc[None, :]
    b_fc2_r = b_fc2[None, :]

    degf = _sc_deg(dst_all, zacc, ones)
    y1 = _tc1(x_all, W_init, b_init_r, W1)
    s1 = _sc_agg(y1, src_all, dst_all, zacc)
    deg3 = degf.reshape(2, N, H)
    y2 = _tc2(y1.reshape(2, N, H), s1.reshape(2, N, H), deg3,
              b1_r, bn1_g_r, bn1_b_r, W2)
    s2 = _sc_agg(y2.reshape(2 * N, H), src_all, dst_all, zacc)
    out = _tc3(y2, s2.reshape(2, N, H), deg3,
               b2_r, bn2_g_r, bn2_b_r, W_fc, b_fc_r, W_fc2, b_fc2_r)
    return out


# speedup vs baseline: 6.7706x; 2.3094x over previous
import functools

import jax
import jax.numpy as jnp
from jax import lax
from jax.experimental import pallas as pl
from jax.experimental.pallas import tpu as pltpu
from jax.experimental.pallas import tpu_sc as plsc

N = 10000
E = 320000
DIN = 128
H = 128

_NC = 2
_NS = 16
_C = 128
_K = 160
_EP = _NS * _K * _C
_NPAD = 10112
_RPT = _NPAD // _NS
_ORT = 624
_TAIL = N - _NS * _ORT

_sc_mesh = plsc.VectorSubcoreMesh(core_axis_name="c", subcore_axis_name="s")


def _sc_deg_body(dst_hbm, zacc_hbm, ones_hbm, deg_hbm,
                 da, db, ones_v, acc_sh, semia, semib, semsa, semsb):
    c = lax.axis_index("c")
    s = lax.axis_index("s")
    tile = c * _NS + s
    ebase = tile * _K * _C

    def idx_start(j, dref, sem):
        pltpu.async_copy(dst_hbm.at[pl.ds(ebase + j * _C, _C)], dref, sem)

    def idx_wait(dref, sem):
        pltpu.make_async_copy(dst_hbm.at[pl.ds(ebase, _C)], dref, sem).wait()

    pltpu.sync_copy(zacc_hbm, acc_sh.at[pl.ds(s * _RPT, _RPT)])
    pltpu.sync_copy(ones_hbm, ones_v)
    idx_start(0, da, semia)
    idx_start(1, db, semib)
    plsc.subcore_barrier()

    def body(i, carry):
        ja = 2 * i
        idx_wait(da, semia)
        pltpu.async_copy(ones_v, acc_sh.at[da], semsa, add=True)
        idx_wait(db, semib)
        pltpu.async_copy(ones_v, acc_sh.at[db], semsb, add=True)
        pltpu.make_async_copy(ones_v, acc_sh.at[da], semsa).wait()

        @pl.when(ja + 2 < _K)
        def _():
            idx_start(ja + 2, da, semia)

        pltpu.make_async_copy(ones_v, acc_sh.at[db], semsb).wait()

        @pl.when(ja + 3 < _K)
        def _():
            idx_start(ja + 3, db, semib)

        return carry

    lax.fori_loop(0, _K // 2, body, 0)
    plsc.subcore_barrier()
    base = c * N + s * _ORT
    pltpu.sync_copy(acc_sh.at[pl.ds(s * _ORT, _ORT)],
                    deg_hbm.at[pl.ds(base, _ORT)])

    @pl.when(s == _NS - 1)
    def _copy_tail():
        pltpu.sync_copy(acc_sh.at[pl.ds(_NS * _ORT, _TAIL)],
                        deg_hbm.at[pl.ds(c * N + _NS * _ORT, _TAIL)])


def _sc_agg_body(y_hbm, src_hbm, dst_hbm, zacc_hbm,
                 out_hbm,
                 sa, da, sb, db, ra, rb, acc_sh,
                 semia, semib, semga, semgb, semsa, semsb):
    c = lax.axis_index("c")
    s = lax.axis_index("s")
    tile = c * _NS + s
    ebase = tile * _K * _C

    def idx_start(j, sref, dref, sem):
        pltpu.async_copy(src_hbm.at[pl.ds(ebase + j * _C, _C)], sref, sem)
        pltpu.async_copy(dst_hbm.at[pl.ds(ebase + j * _C, _C)], dref, sem)

    def idx_wait(sref, dref, sem):
        pltpu.make_async_copy(src_hbm.at[pl.ds(ebase, _C)], sref, sem).wait()
        pltpu.make_async_copy(dst_hbm.at[pl.ds(ebase, _C)], dref, sem).wait()

    pltpu.sync_copy(zacc_hbm, acc_sh.at[pl.ds(s * _RPT, _RPT)])
    idx_start(0, sa, da, semia)
    idx_start(1, sb, db, semib)
    plsc.subcore_barrier()

    def body(i, carry):
        ja = 2 * i
        idx_wait(sa, da, semia)
        pltpu.async_copy(y_hbm.at[sa], ra, semga)
        idx_wait(sb, db, semib)
        pltpu.async_copy(y_hbm.at[sb], rb, semgb)
        pltpu.make_async_copy(y_hbm.at[sa], ra, semga).wait()
        pltpu.async_copy(ra, acc_sh.at[da], semsa, add=True)
        pltpu.make_async_copy(y_hbm.at[sb], rb, semgb).wait()
        pltpu.async_copy(rb, acc_sh.at[db], semsb, add=True)
        pltpu.make_async_copy(ra, acc_sh.at[da], semsa).wait()

        @pl.when(ja + 2 < _K)
        def _():
            idx_start(ja + 2, sa, da, semia)

        pltpu.make_async_copy(rb, acc_sh.at[db], semsb).wait()

        @pl.when(ja + 3 < _K)
        def _():
            idx_start(ja + 3, sb, db, semib)

        return carry

    lax.fori_loop(0, _K // 2, body, 0)
    plsc.subcore_barrier()
    base = c * N + s * _ORT
    pltpu.sync_copy(acc_sh.at[pl.ds(s * _ORT, _ORT)],
                    out_hbm.at[pl.ds(base, _ORT)])

    @pl.when(s == _NS - 1)
    def _copy_tail():
        pltpu.sync_copy(acc_sh.at[pl.ds(_NS * _ORT, _TAIL)],
                        out_hbm.at[pl.ds(c * N + _NS * _ORT, _TAIL)])


_sc_deg = pl.kernel(
    _sc_deg_body,
    out_type=jax.ShapeDtypeStruct((2 * N, H), jnp.float32),
    mesh=_sc_mesh,
    scratch_types=[
        pltpu.VMEM((_C,), jnp.int32),
        pltpu.VMEM((_C,), jnp.int32),
        pltpu.VMEM((_C, H), jnp.float32),
        pltpu.VMEM_SHARED((_NPAD, H), jnp.float32),
        pltpu.SemaphoreType.DMA,
        pltpu.SemaphoreType.DMA,
        pltpu.SemaphoreType.DMA,
        pltpu.SemaphoreType.DMA,
    ],
)

_sc_agg = pl.kernel(
    _sc_agg_body,
    out_type=jax.ShapeDtypeStruct((2 * N, H), jnp.float32),
    mesh=_sc_mesh,
    scratch_types=[
        pltpu.VMEM((_C,), jnp.int32),
        pltpu.VMEM((_C,), jnp.int32),
        pltpu.VMEM((_C,), jnp.int32),
        pltpu.VMEM((_C,), jnp.int32),
        pltpu.VMEM((_C, H), jnp.float32),
        pltpu.VMEM((_C, H), jnp.float32),
        pltpu.VMEM_SHARED((_NPAD, H), jnp.float32),
        pltpu.SemaphoreType.DMA,
        pltpu.SemaphoreType.DMA,
        pltpu.SemaphoreType.DMA,
        pltpu.SemaphoreType.DMA,
        pltpu.SemaphoreType.DMA,
        pltpu.SemaphoreType.DMA,
    ],
)


def _tc1_body(x_ref, wi_ref, bi_ref, w1_ref, y_ref):
    h0 = jnp.dot(x_ref[...], wi_ref[...],
                 preferred_element_type=jnp.float32) + bi_ref[...]
    y_ref[...] = jnp.dot(h0, w1_ref[...], preferred_element_type=jnp.float32)


def _tc1(x_all, w_init, b_init, w1):
    blk = 2000
    grid = (2 * N) // blk
    return pl.pallas_call(
        _tc1_body,
        grid=(grid,),
        in_specs=[
            pl.BlockSpec((blk, DIN), lambda g: (g, 0)),
            pl.BlockSpec((DIN, H), lambda g: (0, 0)),
            pl.BlockSpec((1, H), lambda g: (0, 0)),
            pl.BlockSpec((H, H), lambda g: (0, 0)),
        ],
        out_specs=pl.BlockSpec((blk, H), lambda g: (g, 0)),
        out_shape=jax.ShapeDtypeStruct((2 * N, H), jnp.float32),
    )(x_all, w_init, b_init, w1)


def _bn_relu(z, gamma, beta):
    m = jnp.mean(z, axis=0, keepdims=True)
    v = jnp.mean((z - m) ** 2, axis=0, keepdims=True)
    return jnp.maximum(gamma * (z - m) * lax.rsqrt(v + 1e-5) + beta, 0.0)


def _tc2_body(y_ref, s_ref, deg_ref, b_ref, g_ref, bb_ref, w2_ref, y2_ref):
    inv = 1.0 / jnp.maximum(deg_ref[0, :, 0:1], 1.0)
    z = y_ref[0] + s_ref[0] * inv + b_ref[...]
    h = _bn_relu(z, g_ref[...], bb_ref[...])
    y2_ref[0] = jnp.dot(h, w2_ref[...], preferred_element_type=jnp.float32)


def _tc2(y1, s1, deg, b1, g1, bb1, w2):
    return pl.pallas_call(
        _tc2_body,
        grid=(2,),
        in_specs=[
            pl.BlockSpec((1, N, H), lambda g: (g, 0, 0)),
            pl.BlockSpec((1, N, H), lambda g: (g, 0, 0)),
            pl.BlockSpec((1, N, H), lambda g: (g, 0, 0)),
            pl.BlockSpec((1, H), lambda g: (0, 0)),
            pl.BlockSpec((1, H), lambda g: (0, 0)),
            pl.BlockSpec((1, H), lambda g: (0, 0)),
            pl.BlockSpec((H, H), lambda g: (0, 0)),
        ],
        out_specs=pl.BlockSpec((1, N, H), lambda g: (g, 0, 0)),
        out_shape=jax.ShapeDtypeStruct((2, N, H), jnp.float32),
    )(y1, s1, deg, b1, g1, bb1, w2)


def _tc3_body(y_ref, s_ref, deg_ref, b_ref, g_ref, bb_ref,
              wfc_ref, bfc_ref, wfc2_ref, bfc2_ref, out_ref):
    embs = []
    for g in range(2):
        inv = 1.0 / jnp.maximum(deg_ref[g, :, 0:1], 1.0)
        z = y_ref[g] + s_ref[g] * inv + b_ref[...]
        h = _bn_relu(z, g_ref[...], bb_ref[...])
        embs.append(jnp.mean(h, axis=0, keepdims=True))
    cat = jnp.concatenate(embs, axis=1)
    t = jnp.maximum(jnp.dot(cat, wfc_ref[...],
                            preferred_element_type=jnp.float32) + bfc_ref[...],
                    0.0)
    out_ref[...] = jax.nn.sigmoid(
        jnp.dot(t, wfc2_ref[...], preferred_element_type=jnp.float32)
        + bfc2_ref[...])


def _tc3(y2, s2, deg, b2, g2, bb2, w_fc, b_fc, w_fc2, b_fc2):
    return pl.pallas_call(
        _tc3_body,
        out_shape=jax.ShapeDtypeStruct((1, 1), jnp.float32),
    )(y2, s2, deg, b2, g2, bb2, w_fc, b_fc, w_fc2, b_fc2)


def _prep_edges(edge_index, offset):
    src = edge_index[0]
    dst = edge_index[1]
    pad = _EP - E
    ar = jnp.arange(pad, dtype=jnp.int32)
    srcp = jnp.concatenate([src + offset, ar % N + offset])
    dstp = jnp.concatenate([dst, ar % (_NPAD - N) + N])
    return srcp, dstp


def kernel(x1, edge_index1, x2, edge_index2, W_init, b_init, W1, b1, W2, b2,
           bn1_g, bn1_b, bn2_g, bn2_b, W_fc, b_fc, W_fc2, b_fc2):
    x_all = jnp.concatenate([x1, x2], axis=0)
    src1, dst1 = _prep_edges(edge_index1, 0)
    src2, dst2 = _prep_edges(edge_index2, N)
    src_all = jnp.concatenate([src1, src2])
    dst_all = jnp.concatenate([dst1, dst2])

    zacc = jnp.zeros((_RPT, H), jnp.float32)
    ones = jnp.ones((_C, H), jnp.float32)

    b_init_r = b_init[None, :]
    b1_r = b1[None, :]
    b2_r = b2[None, :]
    bn1_g_r = bn1_g[None, :]
    bn1_b_r = bn1_b[None, :]
    bn2_g_r = bn2_g[None, :]
    bn2_b_r = bn2_b[None, :]
    b_fc_r = b_fc[None, :]
    b_fc2_r = b_fc2[None, :]

    degf = _sc_deg(dst_all, zacc, ones)
    y1 = _tc1(x_all, W_init, b_init_r, W1)
    s1 = _sc_agg(y1, src_all, dst_all, zacc)
    deg3 = degf.reshape(2, N, H)
    y2 = _tc2(y1.reshape(2, N, H), s1.reshape(2, N, H), deg3,
              b1_r, bn1_g_r, bn1_b_r, W2)
    s2 = _sc_agg(y2.reshape(2 * N, H), src_all, dst_all, zacc)
    out = _tc3(y2, s2.reshape(2, N, H), deg3,
               b2_r, bn2_g_r, bn2_b_r, W_fc, b_fc_r, W_fc2, b_fc2_r)
    return out
